# Initial kernel scaffold; baseline (speedup 1.0000x reference)
import jax
import jax.numpy as jnp
from jax.experimental import pallas as pl


def kernel(x, adj, W1, b1, W2, b2, W3, b3, W4, b4, W5, b5, W6, b6):
    raise NotImplementedError("write your pallas kernel here")


# speedup vs baseline: 1.2906x; 1.2906x over previous
import functools

import jax
import jax.numpy as jnp
from jax.experimental import pallas as pl
from jax.experimental.pallas import tpu as pltpu

_PW = 16


def _pad_w(W):
    din, dout = W.shape
    return jnp.zeros((_PW, _PW), W.dtype).at[:din, :dout].set(W)


def _pad_b(b):
    return jnp.zeros((1, _PW), b.dtype).at[0, : b.shape[0]].set(b)


def _layer1_body(x_ref, adj_ref, w_ref, b_ref, g_ref, adjb_ref, s_scr):
    i = pl.program_id(0)

    @pl.when(i == 0)
    def _():
        s_scr[...] = jnp.dot(
            x_ref[...].astype(jnp.bfloat16),
            w_ref[...].astype(jnp.bfloat16),
            preferred_element_type=jnp.float32,
        ).astype(jnp.bfloat16)

    ab = adj_ref[...].astype(jnp.bfloat16)
    adjb_ref[...] = ab
    g_ref[...] = (
        jnp.dot(ab, s_scr[...], preferred_element_type=jnp.float32) + b_ref[...]
    )


def _layers26_body(adjb_ref, g1_ref, w_ref, b_ref, out_ref, g_scr, s_scr, *, blk, nout):
    l = pl.program_id(0)
    i = pl.program_id(1)

    @pl.when(i == 0)
    def _():
        prev = jnp.where(l == 0, g1_ref[...], g_scr[(l + 1) % 2])
        relu = jnp.maximum(prev, 0.0)
        tsh = prev - jnp.tanh(prev)
        act = jnp.where(l == 1, relu, jnp.where((l == 2) | (l == 3), tsh, prev))
        s_scr[...] = jnp.dot(
            act.astype(jnp.bfloat16),
            w_ref[0].astype(jnp.bfloat16),
            preferred_element_type=jnp.float32,
        ).astype(jnp.bfloat16)

    g = jnp.dot(adjb_ref[...], s_scr[...], preferred_element_type=jnp.float32) + b_ref[0]
    g_scr[l % 2, pl.ds(i * blk, blk), :] = g

    @pl.when(l == 4)
    def _():
        out_ref[pl.ds(i * blk, blk), :] = g[:, :nout]


def _pick_block(n):
    for blk in (400, 200, 100, 40, 8):
        if n % blk == 0:
            return blk
    return n


def kernel(x, adj, W1, b1, W2, b2, W3, b3, W4, b4, W5, b5, W6, b6):
    n, feat = x.shape
    nout = W6.shape[1]
    blk = _pick_block(n)
    ni = n // blk

    w1p = jnp.zeros((feat, _PW), W1.dtype).at[:, : W1.shape[1]].set(W1)
    b1p = _pad_b(b1)
    wsp = jnp.stack([_pad_w(W) for W in (W2, W3, W4, W5, W6)])
    bsp = jnp.stack([_pad_b(b) for b in (b2, b3, b4, b5, b6)])

    g1, adjb = pl.pallas_call(
        _layer1_body,
        grid=(ni,),
        in_specs=[
            pl.BlockSpec((n, feat), lambda i: (0, 0)),
            pl.BlockSpec((blk, n), lambda i: (i, 0)),
            pl.BlockSpec((feat, _PW), lambda i: (0, 0)),
            pl.BlockSpec((1, _PW), lambda i: (0, 0)),
        ],
        out_specs=[
            pl.BlockSpec((blk, _PW), lambda i: (i, 0)),
            pl.BlockSpec((blk, n), lambda i: (i, 0)),
        ],
        out_shape=[
            jax.ShapeDtypeStruct((n, _PW), jnp.float32),
            jax.ShapeDtypeStruct((n, n), jnp.bfloat16),
        ],
        scratch_shapes=[pltpu.VMEM((n, _PW), jnp.bfloat16)],
        compiler_params=pltpu.CompilerParams(dimension_semantics=("arbitrary",)),
    )(x, adj, w1p, b1p)

    out = pl.pallas_call(
        functools.partial(_layers26_body, blk=blk, nout=nout),
        grid=(5, ni),
        in_specs=[
            pl.BlockSpec((blk, n), lambda l, i: (i, 0)),
            pl.BlockSpec((n, _PW), lambda l, i: (0, 0)),
            pl.BlockSpec((1, _PW, _PW), lambda l, i: (l, 0, 0)),
            pl.BlockSpec((1, 1, _PW), lambda l, i: (l, 0, 0)),
        ],
        out_specs=pl.BlockSpec((n, nout), lambda l, i: (0, 0)),
        out_shape=jax.ShapeDtypeStruct((n, nout), jnp.float32),
        scratch_shapes=[
            pltpu.VMEM((2, n, _PW), jnp.float32),
            pltpu.VMEM((n, _PW), jnp.bfloat16),
        ],
        compiler_params=pltpu.CompilerParams(
            dimension_semantics=("arbitrary", "arbitrary")
        ),
    )(adjb, g1, wsp, bsp)

    return out


# speedup vs baseline: 1.3821x; 1.0709x over previous
import functools

import jax
import jax.numpy as jnp
from jax.experimental import pallas as pl
from jax.experimental.pallas import tpu as pltpu

_PW = 16


def _pad_w(W):
    din, dout = W.shape
    return jnp.zeros((_PW, _PW), W.dtype).at[:din, :dout].set(W)


def _pad_b(b):
    return jnp.zeros((1, _PW), b.dtype).at[0, : b.shape[0]].set(b)


def _layer1_body(x_ref, adj_ref, w_ref, b_ref, g_ref, qadj_ref, rowinv_ref, s_scr):
    i = pl.program_id(0)

    @pl.when(i == 0)
    def _():
        s_scr[...] = jnp.dot(
            x_ref[...].astype(jnp.bfloat16),
            w_ref[...].astype(jnp.bfloat16),
            preferred_element_type=jnp.float32,
        ).astype(jnp.bfloat16)

    a32 = adj_ref[...]
    rowmax = jnp.maximum(jnp.max(a32, axis=1, keepdims=True), 1e-30)
    qadj_ref[...] = jnp.round(a32 * (127.0 / rowmax)).astype(jnp.int8)
    rowinv_ref[...] = rowmax * (1.0 / 127.0)
    g_ref[...] = (
        jnp.dot(a32.astype(jnp.bfloat16), s_scr[...], preferred_element_type=jnp.float32)
        + b_ref[...]
    )


def _layers26_body(
    qadj_ref, rowinv_ref, g1_ref, w_ref, b_ref, out_ref, g_scr, sq_scr, cinv_scr,
    *, blk, n, nout
):
    l = pl.program_id(0)
    i = pl.program_id(1)

    @pl.when(i == 0)
    def _():
        prev = jnp.where(l == 0, g1_ref[...], g_scr[(l + 1) % 2, :n, :])
        act = jax.lax.switch(
            l,
            [
                lambda v: v,
                lambda v: jnp.maximum(v, 0.0),
                lambda v: v - jnp.tanh(v),
                lambda v: v - jnp.tanh(v),
                lambda v: v,
            ],
            prev,
        )
        s = jnp.dot(
            act.astype(jnp.bfloat16),
            w_ref[0].astype(jnp.bfloat16),
            preferred_element_type=jnp.float32,
        )
        cmax = jnp.maximum(jnp.max(jnp.abs(s), axis=0, keepdims=True), 1e-30)
        sq_scr[...] = jnp.round(s * (127.0 / cmax)).astype(jnp.int8)
        cinv_scr[...] = cmax * (1.0 / 127.0)

    acc = jnp.dot(qadj_ref[...], sq_scr[...], preferred_element_type=jnp.int32)
    g = acc.astype(jnp.float32) * rowinv_ref[...] * cinv_scr[...] + b_ref[0]
    g_scr[l % 2, pl.ds(i * blk, blk), :] = g

    @pl.when(l == 4)
    def _():
        out_ref[pl.ds(i * blk, blk), :] = g[:, :nout]


def _pick_block(n):
    for blk in (400, 200, 100, 40, 8):
        if n % blk == 0:
            return blk
    return n


def kernel(x, adj, W1, b1, W2, b2, W3, b3, W4, b4, W5, b5, W6, b6):
    n, feat = x.shape
    nout = W6.shape[1]
    blk = _pick_block(n)
    ni = n // blk

    w1p = jnp.zeros((feat, _PW), W1.dtype).at[:, : W1.shape[1]].set(W1)
    b1p = _pad_b(b1)
    wsp = jnp.stack([_pad_w(W) for W in (W2, W3, W4, W5, W6)])
    bsp = jnp.stack([_pad_b(b) for b in (b2, b3, b4, b5, b6)])

    g1, qadj, rowinv = pl.pallas_call(
        _layer1_body,
        grid=(ni,),
        in_specs=[
            pl.BlockSpec((n, feat), lambda i: (0, 0)),
            pl.BlockSpec((blk, n), lambda i: (i, 0)),
            pl.BlockSpec((feat, _PW), lambda i: (0, 0)),
            pl.BlockSpec((1, _PW), lambda i: (0, 0)),
        ],
        out_specs=[
            pl.BlockSpec((blk, _PW), lambda i: (i, 0)),
            pl.BlockSpec((blk, n), lambda i: (i, 0)),
            pl.BlockSpec((blk, 1), lambda i: (i, 0)),
        ],
        out_shape=[
            jax.ShapeDtypeStruct((n, _PW), jnp.float32),
            jax.ShapeDtypeStruct((n, n), jnp.int8),
            jax.ShapeDtypeStruct((n, 1), jnp.float32),
        ],
        scratch_shapes=[pltpu.VMEM((n, _PW), jnp.bfloat16)],
        compiler_params=pltpu.CompilerParams(dimension_semantics=("arbitrary",)),
    )(x, adj, w1p, b1p)

    out = pl.pallas_call(
        functools.partial(_layers26_body, blk=blk, n=n, nout=nout),
        grid=(5, ni),
        in_specs=[
            pl.BlockSpec((blk, n), lambda l, i: (i, 0)),
            pl.BlockSpec((blk, 1), lambda l, i: (i, 0)),
            pl.BlockSpec((n, _PW), lambda l, i: (0, 0)),
            pl.BlockSpec((1, _PW, _PW), lambda l, i: (l, 0, 0)),
            pl.BlockSpec((1, 1, _PW), lambda l, i: (l, 0, 0)),
        ],
        out_specs=pl.BlockSpec((n, nout), lambda l, i: (0, 0)),
        out_shape=jax.ShapeDtypeStruct((n, nout), jnp.float32),
        scratch_shapes=[
            pltpu.VMEM((2, n, _PW), jnp.float32),
            pltpu.VMEM((n, _PW), jnp.int8),
            pltpu.VMEM((1, _PW), jnp.float32),
        ],
        compiler_params=pltpu.CompilerParams(
            dimension_semantics=("arbitrary", "arbitrary")
        ),
    )(qadj, rowinv, g1, wsp, bsp)

    return out


# speedup vs baseline: 1.5446x; 1.1176x over previous
import functools

import jax
import jax.numpy as jnp
from jax.experimental import pallas as pl
from jax.experimental.pallas import tpu as pltpu

_PW = 16


def _pad_wt(W):
    din, dout = W.shape
    return jnp.zeros((_PW, _PW), W.dtype).at[:dout, :din].set(W.T)


def _pad_bt(b):
    return jnp.zeros((_PW, 1), b.dtype).at[: b.shape[0], 0].set(b)


def _layer1_body(x_ref, adj_ref, w_ref, b_ref, g_ref, qadj_ref, rowinv_ref, s_scr):
    i = pl.program_id(0)

    @pl.when(i == 0)
    def _():
        s_scr[...] = jnp.dot(
            x_ref[...].astype(jnp.bfloat16),
            w_ref[...].astype(jnp.bfloat16),
            preferred_element_type=jnp.float32,
        ).astype(jnp.bfloat16)

    a32 = adj_ref[...]
    rowmax = jnp.maximum(jnp.max(a32, axis=1, keepdims=True), 1e-30)
    qadj_ref[...] = jnp.round(a32 * (127.0 / rowmax)).astype(jnp.int8)
    rowinv_ref[...] = rowmax * (1.0 / 127.0)
    g_ref[...] = (
        jnp.dot(a32.astype(jnp.bfloat16), s_scr[...], preferred_element_type=jnp.float32)
        + b_ref[...]
    )


def _layers26_body(
    qadj_ref, rowinvT_ref, g1T_ref, wT_ref, bT_ref, outT_ref, g_scr, sq_scr, cinv_scr,
    *, blk, n, nout
):
    l = pl.program_id(0)
    i = pl.program_id(1)

    @pl.when(i == 0)
    def _():
        prev = jnp.where(l == 0, g1T_ref[...], g_scr[(l + 1) % 2, :, :n])
        act = jax.lax.switch(
            l,
            [
                lambda v: v,
                lambda v: jnp.maximum(v, 0.0),
                lambda v: v - jnp.tanh(v),
                lambda v: v - jnp.tanh(v),
                lambda v: v,
            ],
            prev,
        )
        sT = jnp.dot(
            wT_ref[0].astype(jnp.bfloat16),
            act.astype(jnp.bfloat16),
            preferred_element_type=jnp.float32,
        )
        cmax = jnp.maximum(jnp.max(jnp.abs(sT), axis=1, keepdims=True), 1e-30)
        sq_scr[...] = jnp.round(sT * (127.0 / cmax)).astype(jnp.int8)
        cinv_scr[...] = cmax * (1.0 / 127.0)

    acc = jax.lax.dot_general(
        sq_scr[...],
        qadj_ref[...],
        (((1,), (1,)), ((), ())),
        preferred_element_type=jnp.int32,
    )
    g = acc.astype(jnp.float32) * cinv_scr[...] * rowinvT_ref[...] + bT_ref[0]
    g_scr[l % 2, :, pl.ds(i * blk, blk)] = g

    @pl.when(l == 4)
    def _():
        outT_ref[:, pl.ds(i * blk, blk)] = g[:nout, :]


def _pick_block(n):
    for blk in (400, 200, 100, 40, 8):
        if n % blk == 0:
            return blk
    return n


def kernel(x, adj, W1, b1, W2, b2, W3, b3, W4, b4, W5, b5, W6, b6):
    n, feat = x.shape
    nout = W6.shape[1]
    blk1 = _pick_block(n)
    ni1 = n // blk1
    blk2 = 512
    ni2 = -(-n // blk2)
    npad = ni2 * blk2

    w1p = jnp.zeros((feat, _PW), W1.dtype).at[:, : W1.shape[1]].set(W1)
    b1p = jnp.zeros((1, _PW), b1.dtype).at[0, : b1.shape[0]].set(b1)
    wtp = jnp.stack([_pad_wt(W) for W in (W2, W3, W4, W5, W6)])
    btp = jnp.stack([_pad_bt(b) for b in (b2, b3, b4, b5, b6)])

    g1, qadj, rowinv = pl.pallas_call(
        _layer1_body,
        grid=(ni1,),
        in_specs=[
            pl.BlockSpec((n, feat), lambda i: (0, 0)),
            pl.BlockSpec((blk1, n), lambda i: (i, 0)),
            pl.BlockSpec((feat, _PW), lambda i: (0, 0)),
            pl.BlockSpec((1, _PW), lambda i: (0, 0)),
        ],
        out_specs=[
            pl.BlockSpec((blk1, _PW), lambda i: (i, 0)),
            pl.BlockSpec((blk1, n), lambda i: (i, 0)),
            pl.BlockSpec((blk1, 1), lambda i: (i, 0)),
        ],
        out_shape=[
            jax.ShapeDtypeStruct((n, _PW), jnp.float32),
            jax.ShapeDtypeStruct((n, n), jnp.int8),
            jax.ShapeDtypeStruct((n, 1), jnp.float32),
        ],
        scratch_shapes=[pltpu.VMEM((n, _PW), jnp.bfloat16)],
        compiler_params=pltpu.CompilerParams(dimension_semantics=("arbitrary",)),
    )(x, adj, w1p, b1p)

    g1T = g1.T
    rowinvT = rowinv.reshape(1, n)

    outT = pl.pallas_call(
        functools.partial(_layers26_body, blk=blk2, n=n, nout=nout),
        grid=(5, ni2),
        in_specs=[
            pl.BlockSpec((blk2, n), lambda l, i: (i, 0)),
            pl.BlockSpec((1, blk2), lambda l, i: (0, i)),
            pl.BlockSpec((_PW, n), lambda l, i: (0, 0)),
            pl.BlockSpec((1, _PW, _PW), lambda l, i: (l, 0, 0)),
            pl.BlockSpec((1, _PW, 1), lambda l, i: (l, 0, 0)),
        ],
        out_specs=pl.BlockSpec((nout, npad), lambda l, i: (0, 0)),
        out_shape=jax.ShapeDtypeStruct((nout, npad), jnp.float32),
        scratch_shapes=[
            pltpu.VMEM((2, _PW, npad), jnp.float32),
            pltpu.VMEM((_PW, n), jnp.int8),
            pltpu.VMEM((_PW, 1), jnp.float32),
        ],
        compiler_params=pltpu.CompilerParams(
            dimension_semantics=("arbitrary", "arbitrary")
        ),
    )(qadj, rowinvT, g1T, wtp, btp)

    return outT[:, :n].T


# speedup vs baseline: 1.5887x; 1.0285x over previous
import functools

import jax
import jax.numpy as jnp
from jax.experimental import pallas as pl
from jax.experimental.pallas import tpu as pltpu

_PW = 16


def _pad_wt(W):
    din, dout = W.shape
    return jnp.zeros((_PW, _PW), W.dtype).at[:dout, :din].set(W.T)


def _pad_bt(b):
    return jnp.zeros((_PW, 1), b.dtype).at[: b.shape[0], 0].set(b)


def _layer1_body(x_ref, adj_ref, wt_ref, bt_ref, gt_ref, qadj_ref, rowinv_ref, s_scr):
    i = pl.program_id(0)

    @pl.when(i == 0)
    def _():
        s_scr[...] = jax.lax.dot_general(
            wt_ref[...],
            x_ref[...],
            (((1,), (1,)), ((), ())),
            preferred_element_type=jnp.float32,
        )

    a32 = adj_ref[...]
    rowmax = jnp.maximum(jnp.max(a32, axis=1, keepdims=True), 1e-30)
    qadj_ref[...] = jnp.round(a32 * (127.0 / rowmax)).astype(jnp.int8)
    rowinv_ref[...] = rowmax * (1.0 / 127.0)
    gt_ref[...] = (
        jax.lax.dot_general(
            s_scr[...],
            a32,
            (((1,), (1,)), ((), ())),
            preferred_element_type=jnp.float32,
        )
        + bt_ref[...]
    )


def _layers26_body(
    qadj_ref, rowinvT_ref, g1T_ref, wT_ref, bT_ref, outT_ref, g_scr, sq_scr, cinv_scr,
    *, blk, n, nout
):
    l = pl.program_id(0)
    i = pl.program_id(1)

    @pl.when(i == 0)
    def _():
        prev = jnp.where(l == 0, g1T_ref[...], g_scr[(l + 1) % 2, :, :n])
        act = jax.lax.switch(
            l,
            [
                lambda v: v,
                lambda v: jnp.maximum(v, 0.0),
                lambda v: v - jnp.tanh(v),
                lambda v: v - jnp.tanh(v),
                lambda v: v,
            ],
            prev,
        )
        sT = jnp.dot(
            wT_ref[0].astype(jnp.bfloat16),
            act.astype(jnp.bfloat16),
            preferred_element_type=jnp.float32,
        )
        cmax = jnp.maximum(jnp.max(jnp.abs(sT), axis=1, keepdims=True), 1e-30)
        sq_scr[...] = jnp.round(sT * (127.0 / cmax)).astype(jnp.int8)
        cinv_scr[...] = cmax * (1.0 / 127.0)

    acc = jax.lax.dot_general(
        sq_scr[...],
        qadj_ref[...],
        (((1,), (1,)), ((), ())),
        preferred_element_type=jnp.int32,
    )
    g = acc.astype(jnp.float32) * cinv_scr[...] * rowinvT_ref[...] + bT_ref[0]
    g_scr[l % 2, :, pl.ds(i * blk, blk)] = g

    @pl.when(l == 4)
    def _():
        outT_ref[:, pl.ds(i * blk, blk)] = g[:nout, :]


def _pick_block(n):
    for blk in (400, 200, 100, 40, 8):
        if n % blk == 0:
            return blk
    return n


def kernel(x, adj, W1, b1, W2, b2, W3, b3, W4, b4, W5, b5, W6, b6):
    n, feat = x.shape
    nout = W6.shape[1]
    blk1 = 512
    ni1 = -(-n // blk1)
    blk2 = 2048
    ni2 = -(-n // blk2)
    npad = ni2 * blk2

    w1tp = jnp.zeros((_PW, feat), W1.dtype).at[: W1.shape[1], :].set(W1.T)
    b1tp = _pad_bt(b1)
    wtp = jnp.stack([_pad_wt(W) for W in (W2, W3, W4, W5, W6)])
    btp = jnp.stack([_pad_bt(b) for b in (b2, b3, b4, b5, b6)])

    g1T, qadj, rowinv = pl.pallas_call(
        _layer1_body,
        grid=(ni1,),
        in_specs=[
            pl.BlockSpec((n, feat), lambda i: (0, 0)),
            pl.BlockSpec((blk1, n), lambda i: (i, 0)),
            pl.BlockSpec((_PW, feat), lambda i: (0, 0)),
            pl.BlockSpec((_PW, 1), lambda i: (0, 0)),
        ],
        out_specs=[
            pl.BlockSpec((_PW, blk1), lambda i: (0, i)),
            pl.BlockSpec((blk1, n), lambda i: (i, 0)),
            pl.BlockSpec((blk1, 1), lambda i: (i, 0)),
        ],
        out_shape=[
            jax.ShapeDtypeStruct((_PW, n), jnp.float32),
            jax.ShapeDtypeStruct((n, n), jnp.int8),
            jax.ShapeDtypeStruct((n, 1), jnp.float32),
        ],
        scratch_shapes=[pltpu.VMEM((_PW, n), jnp.float32)],
        compiler_params=pltpu.CompilerParams(dimension_semantics=("arbitrary",)),
    )(x, adj, w1tp, b1tp)

    rowinvT = rowinv.reshape(1, n)

    outT = pl.pallas_call(
        functools.partial(_layers26_body, blk=blk2, n=n, nout=nout),
        grid=(5, ni2),
        in_specs=[
            pl.BlockSpec((blk2, n), lambda l, i: (i, 0)),
            pl.BlockSpec((1, blk2), lambda l, i: (0, i)),
            pl.BlockSpec((_PW, n), lambda l, i: (0, 0)),
            pl.BlockSpec((1, _PW, _PW), lambda l, i: (l, 0, 0)),
            pl.BlockSpec((1, _PW, 1), lambda l, i: (l, 0, 0)),
        ],
        out_specs=pl.BlockSpec((nout, npad), lambda l, i: (0, 0)),
        out_shape=jax.ShapeDtypeStruct((nout, npad), jnp.float32),
        scratch_shapes=[
            pltpu.VMEM((2, _PW, npad), jnp.float32),
            pltpu.VMEM((_PW, n), jnp.int8),
            pltpu.VMEM((_PW, 1), jnp.float32),
        ],
        compiler_params=pltpu.CompilerParams(
            dimension_semantics=("arbitrary", "arbitrary")
        ),
    )(qadj, rowinvT, g1T, wtp, btp)

    return outT[:, :n].T


# speedup vs baseline: 2.1213x; 1.3353x over previous
import functools

import jax
import jax.numpy as jnp
from jax.experimental import pallas as pl
from jax.experimental.pallas import tpu as pltpu

_PW = 16


def _pad_wt(W):
    din, dout = W.shape
    return jnp.zeros((_PW, _PW), W.dtype).at[:dout, :din].set(W.T)


def _pad_bt(b):
    return jnp.zeros((_PW, 1), b.dtype).at[: b.shape[0], 0].set(b)


def _layer1_body(x_ref, adj_ref, wt_ref, bt_ref, gt_ref, qadj_ref, rowinv_ref, s_scr):
    i = pl.program_id(0)

    @pl.when(i == 0)
    def _():
        s_scr[...] = jax.lax.dot_general(
            wt_ref[...],
            x_ref[...],
            (((1,), (1,)), ((), ())),
            preferred_element_type=jnp.float32,
        )

    a32 = adj_ref[...]
    rowmax = jnp.maximum(jnp.max(a32, axis=1, keepdims=True), 1e-30)
    qadj_ref[...] = (a32 * (256.0 / rowmax)).astype(jnp.float8_e4m3fn)
    rowinv_ref[...] = rowmax * (1.0 / 256.0)
    gt_ref[...] = (
        jax.lax.dot_general(
            s_scr[...],
            a32,
            (((1,), (1,)), ((), ())),
            preferred_element_type=jnp.float32,
        )
        + bt_ref[...]
    )


def _layers26_body(
    qadj_ref, rowinvT_ref, g1T_ref, wT_ref, bT_ref, outT_ref, g_scr, sq_scr, cinv_scr,
    *, blk, n, nout
):
    l = pl.program_id(0)
    i = pl.program_id(1)

    @pl.when(i == 0)
    def _():
        prev = jnp.where(l == 0, g1T_ref[...], g_scr[(l + 1) % 2, :, :n])
        act = jax.lax.switch(
            l,
            [
                lambda v: v,
                lambda v: jnp.maximum(v, 0.0),
                lambda v: v - jnp.tanh(v),
                lambda v: v - jnp.tanh(v),
                lambda v: v,
            ],
            prev,
        )
        sT = jnp.dot(
            wT_ref[0].astype(jnp.bfloat16),
            act.astype(jnp.bfloat16),
            preferred_element_type=jnp.float32,
        )
        cmax = jnp.maximum(jnp.max(jnp.abs(sT), axis=1, keepdims=True), 1e-30)
        sq_scr[...] = (sT * (256.0 / cmax)).astype(jnp.float8_e4m3fn)
        cinv_scr[...] = cmax * (1.0 / 256.0)

    acc = jax.lax.dot_general(
        sq_scr[...],
        qadj_ref[...],
        (((1,), (1,)), ((), ())),
        preferred_element_type=jnp.float32,
    )
    g = acc.astype(jnp.float32) * cinv_scr[...] * rowinvT_ref[...] + bT_ref[0]
    g_scr[l % 2, :, pl.ds(i * blk, blk)] = g

    @pl.when(l == 4)
    def _():
        outT_ref[:, pl.ds(i * blk, blk)] = g[:nout, :]


def _pick_block(n):
    for blk in (400, 200, 100, 40, 8):
        if n % blk == 0:
            return blk
    return n


def kernel(x, adj, W1, b1, W2, b2, W3, b3, W4, b4, W5, b5, W6, b6):
    n, feat = x.shape
    nout = W6.shape[1]
    blk1 = 512
    ni1 = -(-n // blk1)
    blk2 = 2048
    ni2 = -(-n // blk2)
    npad = ni2 * blk2

    w1tp = jnp.zeros((_PW, feat), W1.dtype).at[: W1.shape[1], :].set(W1.T)
    b1tp = _pad_bt(b1)
    wtp = jnp.stack([_pad_wt(W) for W in (W2, W3, W4, W5, W6)])
    btp = jnp.stack([_pad_bt(b) for b in (b2, b3, b4, b5, b6)])

    g1T, qadj, rowinv = pl.pallas_call(
        _layer1_body,
        grid=(ni1,),
        in_specs=[
            pl.BlockSpec((n, feat), lambda i: (0, 0)),
            pl.BlockSpec((blk1, n), lambda i: (i, 0)),
            pl.BlockSpec((_PW, feat), lambda i: (0, 0)),
            pl.BlockSpec((_PW, 1), lambda i: (0, 0)),
        ],
        out_specs=[
            pl.BlockSpec((_PW, blk1), lambda i: (0, i)),
            pl.BlockSpec((blk1, n), lambda i: (i, 0)),
            pl.BlockSpec((blk1, 1), lambda i: (i, 0)),
        ],
        out_shape=[
            jax.ShapeDtypeStruct((_PW, n), jnp.float32),
            jax.ShapeDtypeStruct((n, n), jnp.float8_e4m3fn),
            jax.ShapeDtypeStruct((n, 1), jnp.float32),
        ],
        scratch_shapes=[pltpu.VMEM((_PW, n), jnp.float32)],
        compiler_params=pltpu.CompilerParams(dimension_semantics=("arbitrary",)),
    )(x, adj, w1tp, b1tp)

    rowinvT = rowinv.reshape(1, n)

    outT = pl.pallas_call(
        functools.partial(_layers26_body, blk=blk2, n=n, nout=nout),
        grid=(5, ni2),
        in_specs=[
            pl.BlockSpec((blk2, n), lambda l, i: (i, 0)),
            pl.BlockSpec((1, blk2), lambda l, i: (0, i)),
            pl.BlockSpec((_PW, n), lambda l, i: (0, 0)),
            pl.BlockSpec((1, _PW, _PW), lambda l, i: (l, 0, 0)),
            pl.BlockSpec((1, _PW, 1), lambda l, i: (l, 0, 0)),
        ],
        out_specs=pl.BlockSpec((nout, npad), lambda l, i: (0, 0)),
        out_shape=jax.ShapeDtypeStruct((nout, npad), jnp.float32),
        scratch_shapes=[
            pltpu.VMEM((2, _PW, npad), jnp.float32),
            pltpu.VMEM((_PW, n), jnp.float8_e4m3fn),
            pltpu.VMEM((_PW, 1), jnp.float32),
        ],
        compiler_params=pltpu.CompilerParams(
            dimension_semantics=("arbitrary", "arbitrary")
        ),
    )(qadj, rowinvT, g1T, wtp, btp)

    return outT[:, :n].T


# speedup vs baseline: 2.2031x; 1.0386x over previous
import functools

import jax
import jax.numpy as jnp
from jax.experimental import pallas as pl
from jax.experimental.pallas import tpu as pltpu

_PW = 16


def _pad_wt(W):
    din, dout = W.shape
    return jnp.zeros((_PW, _PW), W.dtype).at[:dout, :din].set(W.T)


def _pad_bt(b):
    return jnp.zeros((_PW, 1), b.dtype).at[: b.shape[0], 0].set(b)


def _layer1_body(x_ref, adj_ref, wt_ref, gt_ref, qadj_ref, rowinv_ref, sq_scr, cinv_scr):
    i = pl.program_id(0)

    @pl.when(i == 0)
    def _():
        s1 = jax.lax.dot_general(
            wt_ref[...],
            x_ref[...],
            (((1,), (1,)), ((), ())),
            preferred_element_type=jnp.float32,
        )
        cmax = jnp.maximum(jnp.max(jnp.abs(s1), axis=1, keepdims=True), 1e-30)
        sq_scr[...] = (s1 * (256.0 / cmax)).astype(jnp.float8_e4m3fn)
        cinv_scr[...] = cmax * (1.0 / 256.0)

    a32 = adj_ref[...]
    rowmax = jnp.maximum(jnp.max(a32, axis=1, keepdims=True), 1e-30)
    q = (a32 * (256.0 / rowmax)).astype(jnp.float8_e4m3fn)
    qadj_ref[...] = q
    rowinv_ref[...] = rowmax * (1.0 / 256.0)
    acc = jax.lax.dot_general(
        sq_scr[...],
        q,
        (((1,), (1,)), ((), ())),
        preferred_element_type=jnp.float32,
    )
    gt_ref[...] = acc * cinv_scr[...]


def _layers26_body(
    qadj_ref, rowinvT_ref, rowinvF_ref, g1T_ref, wT_ref, bT_ref, b1T_ref, outT_ref,
    g_scr, sq_scr, cinv_scr, *, blk, n, nout
):
    l = pl.program_id(0)
    i = pl.program_id(1)

    @pl.when(i == 0)
    def _():
        prev = jnp.where(l == 0, g1T_ref[...], g_scr[(l + 1) % 2, :, :n])
        act = jax.lax.switch(
            l,
            [
                lambda v: v,
                lambda v: jnp.maximum(v, 0.0),
                lambda v: v - jnp.tanh(v),
                lambda v: v - jnp.tanh(v),
                lambda v: v,
            ],
            prev,
        )
        sT = jnp.dot(
            wT_ref[0].astype(jnp.bfloat16),
            act.astype(jnp.bfloat16),
            preferred_element_type=jnp.float32,
        )
        sT = jnp.where(
            l == 0,
            sT * rowinvF_ref[...] + jnp.dot(wT_ref[0], b1T_ref[...]),
            sT,
        )
        cmax = jnp.maximum(jnp.max(jnp.abs(sT), axis=1, keepdims=True), 1e-30)
        sq_scr[...] = (sT * (256.0 / cmax)).astype(jnp.float8_e4m3fn)
        cinv_scr[...] = cmax * (1.0 / 256.0)

    acc = jax.lax.dot_general(
        sq_scr[...],
        qadj_ref[...],
        (((1,), (1,)), ((), ())),
        preferred_element_type=jnp.float32,
    )
    g = acc.astype(jnp.float32) * cinv_scr[...] * rowinvT_ref[...] + bT_ref[0]
    g_scr[l % 2, :, pl.ds(i * blk, blk)] = g

    @pl.when(l == 4)
    def _():
        outT_ref[:, pl.ds(i * blk, blk)] = g[:nout, :]


def _pick_block(n):
    for blk in (400, 200, 100, 40, 8):
        if n % blk == 0:
            return blk
    return n


def kernel(x, adj, W1, b1, W2, b2, W3, b3, W4, b4, W5, b5, W6, b6):
    n, feat = x.shape
    nout = W6.shape[1]
    blk1 = 512
    ni1 = -(-n // blk1)
    blk2 = 2048
    ni2 = -(-n // blk2)
    npad = ni2 * blk2

    w1tp = jnp.zeros((_PW, feat), W1.dtype).at[: W1.shape[1], :].set(W1.T)
    b1tp = _pad_bt(b1)
    wtp = jnp.stack([_pad_wt(W) for W in (W2, W3, W4, W5, W6)])
    btp = jnp.stack([_pad_bt(b) for b in (b2, b3, b4, b5, b6)])

    g1T, qadj, rowinv = pl.pallas_call(
        _layer1_body,
        grid=(ni1,),
        in_specs=[
            pl.BlockSpec((n, feat), lambda i: (0, 0)),
            pl.BlockSpec((blk1, n), lambda i: (i, 0)),
            pl.BlockSpec((_PW, feat), lambda i: (0, 0)),
        ],
        out_specs=[
            pl.BlockSpec((_PW, blk1), lambda i: (0, i)),
            pl.BlockSpec((blk1, n), lambda i: (i, 0)),
            pl.BlockSpec((blk1, 1), lambda i: (i, 0)),
        ],
        out_shape=[
            jax.ShapeDtypeStruct((_PW, n), jnp.float32),
            jax.ShapeDtypeStruct((n, n), jnp.float8_e4m3fn),
            jax.ShapeDtypeStruct((n, 1), jnp.float32),
        ],
        scratch_shapes=[
            pltpu.VMEM((_PW, n), jnp.float8_e4m3fn),
            pltpu.VMEM((_PW, 1), jnp.float32),
        ],
        compiler_params=pltpu.CompilerParams(dimension_semantics=("arbitrary",)),
    )(x, adj, w1tp)

    rowinvT = rowinv.reshape(1, n)

    outT = pl.pallas_call(
        functools.partial(_layers26_body, blk=blk2, n=n, nout=nout),
        grid=(5, ni2),
        in_specs=[
            pl.BlockSpec((blk2, n), lambda l, i: (i, 0)),
            pl.BlockSpec((1, blk2), lambda l, i: (0, i)),
            pl.BlockSpec((1, n), lambda l, i: (0, 0)),
            pl.BlockSpec((_PW, n), lambda l, i: (0, 0)),
            pl.BlockSpec((1, _PW, _PW), lambda l, i: (l, 0, 0)),
            pl.BlockSpec((1, _PW, 1), lambda l, i: (l, 0, 0)),
            pl.BlockSpec((_PW, 1), lambda l, i: (0, 0)),
        ],
        out_specs=pl.BlockSpec((nout, npad), lambda l, i: (0, 0)),
        out_shape=jax.ShapeDtypeStruct((nout, npad), jnp.float32),
        scratch_shapes=[
            pltpu.VMEM((2, _PW, npad), jnp.float32),
            pltpu.VMEM((_PW, n), jnp.float8_e4m3fn),
            pltpu.VMEM((_PW, 1), jnp.float32),
        ],
        compiler_params=pltpu.CompilerParams(
            dimension_semantics=("arbitrary", "arbitrary")
        ),
    )(qadj, rowinvT, rowinvT, g1T, wtp, btp, b1tp)

    return outT[:, :n].T
